# baseline (speedup 1.0000x reference)
import functools
import jax
import jax.numpy as jnp
from jax import lax
from jax.experimental import pallas as pl
from jax.experimental.pallas import tpu as pltpu
from jax.experimental.pallas import tpu_sc as plsc

VOCAB_N = 1000000
D_N = 32
B_N = 16384
L_N = 200

VBLK = 65536
GRID1 = -(-VOCAB_N // VBLK)
SPAD = GRID1 * VBLK


def _matvec_body(t_ref, w_ref, o_ref):
    o_ref[...] = jnp.sum(t_ref[...] * w_ref[:, 0:1], axis=0)


def _stage1(table_t, w128):
    return pl.pallas_call(
        _matvec_body,
        grid=(GRID1,),
        in_specs=[
            pl.BlockSpec((D_N, VBLK), lambda i: (0, i)),
            pl.BlockSpec((D_N, 128), lambda i: (0, 0)),
        ],
        out_specs=pl.BlockSpec((VBLK,), lambda i: (i,)),
        out_shape=jax.ShapeDtypeStruct((SPAD,), jnp.float32),
    )(table_t, w128)


NC = 2
NS = 16
NW = NC * NS
CCH = 128
NCH = (B_N // NW) // CCH
CHUNK = L_N * CCH
LA = 152
LB = L_N - LA
HALFA = LA * CCH
HALFB = LB * CCH
INV = 1.0 / (2 * L_N - 1)

_mesh = plsc.VectorSubcoreMesh(core_axis_name="c", subcore_axis_name="s")


@functools.partial(
    pl.kernel,
    mesh=_mesh,
    out_type=jax.ShapeDtypeStruct((B_N,), jnp.float32),
    scratch_types=[
        pltpu.VMEM((LA, CCH), jnp.int32),
        pltpu.VMEM((LB, CCH), jnp.int32),
        pltpu.VMEM((HALFA,), jnp.float32),
        pltpu.VMEM((HALFB,), jnp.float32),
        pltpu.VMEM((CCH,), jnp.float32),
        pltpu.VMEM((16,), jnp.float32),
        pltpu.VMEM_SHARED((SPAD,), jnp.float32),
        pltpu.SemaphoreType.DMA,
        pltpu.SemaphoreType.DMA,
        pltpu.SemaphoreType.DMA,
    ],
)
def _pool_kernel(xt_hbm, s_hbm, b16_hbm, out_hbm, idx_a, idx_b, vals_a,
                 vals_b, outc_v, b_v, s_sh, sem_a, sem_b, sem_i):
    wid = lax.axis_index("s") * NC + lax.axis_index("c")
    pltpu.sync_copy(b16_hbm, b_v)
    bias = b_v[...]

    def load_idx(ch):
        col = (wid * NCH + ch) * CCH
        ia = pltpu.async_copy(
            xt_hbm.at[pl.ds(0, LA), pl.ds(col, CCH)], idx_a, sem_i)
        ib = pltpu.async_copy(
            xt_hbm.at[pl.ds(LA, LB), pl.ds(col, CCH)], idx_b, sem_i)
        return ia, ib

    def start_gathers():
        ca = pltpu.async_copy(s_sh.at[idx_a.reshape(1, HALFA).at[0]], vals_a,
                              sem_a)
        cb = pltpu.async_copy(s_hbm.at[idx_b.reshape(1, HALFB).at[0]], vals_b,
                              sem_b)
        return ca, cb

    ia, ib = load_idx(0)
    sid = lax.axis_index("s")
    spart = SPAD // NS
    pltpu.sync_copy(s_hbm.at[pl.ds(sid * spart, spart)],
                    s_sh.at[pl.ds(sid * spart, spart)])
    plsc.subcore_barrier()
    ia.wait()
    ca = pltpu.async_copy(s_sh.at[idx_a.reshape(1, HALFA).at[0]], vals_a,
                          sem_a)
    ib.wait()
    cb = pltpu.async_copy(s_hbm.at[idx_b.reshape(1, HALFB).at[0]], vals_b,
                          sem_b)
    for ch in range(NCH):
        ca.wait()
        cb.wait()
        ci = None
        if ch + 1 < NCH:
            ci = load_idx(ch + 1)
        for g in range(CCH // 16):
            off = g * 16

            def body_a(ll, acc, off=off):
                for u in range(4):
                    acc = acc + vals_a[pl.ds((ll * 4 + u) * CCH + off, 16)]
                return acc

            def body_b(ll, acc, off=off):
                for u in range(4):
                    acc = acc + vals_b[pl.ds((ll * 4 + u) * CCH + off, 16)]
                return acc

            acc = lax.fori_loop(0, LA // 4, body_a,
                                jnp.zeros((16,), jnp.float32))
            acc = lax.fori_loop(0, LB // 4, body_b, acc)
            first = vals_a[pl.ds(off, 16)]
            last = vals_b[pl.ds((LB - 1) * CCH + off, 16)]
            outc_v[pl.ds(off, 16)] = (3.0 * acc - first - last) * INV + bias
        blk = wid * NCH + ch
        pltpu.sync_copy(outc_v, out_hbm.at[pl.ds(blk * CCH, CCH)])
        if ci is not None:
            ci[0].wait()
            ci[1].wait()
            ca, cb = start_gathers()


def kernel(x_in, table, W, b):
    table_t = table.T
    w128 = jnp.broadcast_to(W, (D_N, 128))
    s = _stage1(table_t, w128)
    xt = x_in.astype(jnp.int32).T
    b16 = jnp.broadcast_to(b, (16,))
    pooled = _pool_kernel(xt, s, b16)
    return pooled.reshape(B_N, 1)


# speedup vs baseline: 1.0374x; 1.0374x over previous
import functools
import jax
import jax.numpy as jnp
from jax import lax
from jax.experimental import pallas as pl
from jax.experimental.pallas import tpu as pltpu
from jax.experimental.pallas import tpu_sc as plsc

VOCAB_N = 1000000
D_N = 32
B_N = 16384
L_N = 200

VBLK = 65536
GRID1 = -(-VOCAB_N // VBLK)
SPAD = GRID1 * VBLK


def _matvec_body(t_ref, w_ref, o_ref):
    o_ref[...] = jnp.sum(t_ref[...] * w_ref[:, 0:1], axis=0)


def _stage1(table_t, w128):
    return pl.pallas_call(
        _matvec_body,
        grid=(GRID1,),
        in_specs=[
            pl.BlockSpec((D_N, VBLK), lambda i: (0, i)),
            pl.BlockSpec((D_N, 128), lambda i: (0, 0)),
        ],
        out_specs=pl.BlockSpec((VBLK,), lambda i: (i,)),
        out_shape=jax.ShapeDtypeStruct((SPAD,), jnp.float32),
    )(table_t, w128)


NC = 2
NS = 16
NW = NC * NS
CCH = 128
NCH = (B_N // NW) // CCH
CHUNK = L_N * CCH
LA = 96
LB = L_N - LA
HALFA = LA * CCH
HALFB = LB * CCH
INV = 1.0 / (2 * L_N - 1)

_mesh = plsc.VectorSubcoreMesh(core_axis_name="c", subcore_axis_name="s")


@functools.partial(
    pl.kernel,
    mesh=_mesh,
    out_type=jax.ShapeDtypeStruct((B_N,), jnp.float32),
    scratch_types=[
        pltpu.VMEM((LA, CCH), jnp.int32),
        pltpu.VMEM((LB, CCH), jnp.int32),
        pltpu.VMEM((HALFA,), jnp.float32),
        pltpu.VMEM((HALFB,), jnp.float32),
        pltpu.VMEM((CCH,), jnp.float32),
        pltpu.VMEM((16,), jnp.float32),
        pltpu.VMEM_SHARED((SPAD,), jnp.float32),
        pltpu.SemaphoreType.DMA,
        pltpu.SemaphoreType.DMA,
        pltpu.SemaphoreType.DMA,
    ],
)
def _pool_kernel(xt_hbm, s_hbm, b16_hbm, out_hbm, idx_a, idx_b, vals_a,
                 vals_b, outc_v, b_v, s_sh, sem_a, sem_b, sem_i):
    wid = lax.axis_index("s") * NC + lax.axis_index("c")
    pltpu.sync_copy(b16_hbm, b_v)
    bias = b_v[...]

    def load_idx(ch):
        col = (wid * NCH + ch) * CCH
        ia = pltpu.async_copy(
            xt_hbm.at[pl.ds(0, LA), pl.ds(col, CCH)], idx_a, sem_i)
        ib = pltpu.async_copy(
            xt_hbm.at[pl.ds(LA, LB), pl.ds(col, CCH)], idx_b, sem_i)
        return ia, ib

    def start_gathers():
        ca = pltpu.async_copy(s_sh.at[idx_a.reshape(1, HALFA).at[0]], vals_a,
                              sem_a)
        cb = pltpu.async_copy(s_sh.at[idx_b.reshape(1, HALFB).at[0]], vals_b,
                              sem_b)
        return ca, cb

    ia, ib = load_idx(0)
    sid = lax.axis_index("s")
    spart = SPAD // NS
    pltpu.sync_copy(s_hbm.at[pl.ds(sid * spart, spart)],
                    s_sh.at[pl.ds(sid * spart, spart)])
    plsc.subcore_barrier()
    ia.wait()
    ca = pltpu.async_copy(s_sh.at[idx_a.reshape(1, HALFA).at[0]], vals_a,
                          sem_a)
    ib.wait()
    cb = pltpu.async_copy(s_sh.at[idx_b.reshape(1, HALFB).at[0]], vals_b,
                          sem_b)
    for ch in range(NCH):
        ca.wait()
        cb.wait()
        ci = None
        if ch + 1 < NCH:
            ci = load_idx(ch + 1)
        for g in range(CCH // 16):
            off = g * 16

            def body_a(ll, acc, off=off):
                for u in range(4):
                    acc = acc + vals_a[pl.ds((ll * 4 + u) * CCH + off, 16)]
                return acc

            def body_b(ll, acc, off=off):
                for u in range(4):
                    acc = acc + vals_b[pl.ds((ll * 4 + u) * CCH + off, 16)]
                return acc

            acc = lax.fori_loop(0, LA // 4, body_a,
                                jnp.zeros((16,), jnp.float32))
            acc = lax.fori_loop(0, LB // 4, body_b, acc)
            first = vals_a[pl.ds(off, 16)]
            last = vals_b[pl.ds((LB - 1) * CCH + off, 16)]
            outc_v[pl.ds(off, 16)] = (3.0 * acc - first - last) * INV + bias
        blk = wid * NCH + ch
        pltpu.sync_copy(outc_v, out_hbm.at[pl.ds(blk * CCH, CCH)])
        if ci is not None:
            ci[0].wait()
            ci[1].wait()
            ca, cb = start_gathers()


def kernel(x_in, table, W, b):
    table_t = table.T
    w128 = jnp.broadcast_to(W, (D_N, 128))
    s = _stage1(table_t, w128)
    xt = x_in.astype(jnp.int32).T
    b16 = jnp.broadcast_to(b, (16,))
    pooled = _pool_kernel(xt, s, b16)
    return pooled.reshape(B_N, 1)


# speedup vs baseline: 1.0374x; 1.0000x over previous
import functools
import jax
import jax.numpy as jnp
from jax import lax
from jax.experimental import pallas as pl
from jax.experimental.pallas import tpu as pltpu
from jax.experimental.pallas import tpu_sc as plsc

VOCAB_N = 1000000
D_N = 32
B_N = 16384
L_N = 200

VBLK = 131072
GRID1 = -(-VOCAB_N // VBLK)
SPAD = GRID1 * VBLK


def _matvec_body(t_ref, w_ref, o_ref):
    o_ref[...] = jnp.sum(t_ref[...] * w_ref[:, 0:1], axis=0)


def _stage1(table_t, w128):
    return pl.pallas_call(
        _matvec_body,
        grid=(GRID1,),
        in_specs=[
            pl.BlockSpec((D_N, VBLK), lambda i: (0, i)),
            pl.BlockSpec((D_N, 128), lambda i: (0, 0)),
        ],
        out_specs=pl.BlockSpec((VBLK,), lambda i: (i,)),
        out_shape=jax.ShapeDtypeStruct((SPAD,), jnp.float32),
    )(table_t, w128)


NC = 2
NS = 16
NW = NC * NS
CCH = 128
NCH = (B_N // NW) // CCH
CHUNK = L_N * CCH
LA = 96
LB = L_N - LA
HALFA = LA * CCH
HALFB = LB * CCH
INV = 1.0 / (2 * L_N - 1)

_mesh = plsc.VectorSubcoreMesh(core_axis_name="c", subcore_axis_name="s")


@functools.partial(
    pl.kernel,
    mesh=_mesh,
    out_type=jax.ShapeDtypeStruct((B_N,), jnp.float32),
    scratch_types=[
        pltpu.VMEM((LA, CCH), jnp.int32),
        pltpu.VMEM((LB, CCH), jnp.int32),
        pltpu.VMEM((HALFA,), jnp.float32),
        pltpu.VMEM((HALFB,), jnp.float32),
        pltpu.VMEM((CCH,), jnp.float32),
        pltpu.VMEM((16,), jnp.float32),
        pltpu.VMEM_SHARED((SPAD,), jnp.float32),
        pltpu.SemaphoreType.DMA,
        pltpu.SemaphoreType.DMA,
        pltpu.SemaphoreType.DMA,
    ],
)
def _pool_kernel(xt_hbm, s_hbm, b16_hbm, out_hbm, idx_a, idx_b, vals_a,
                 vals_b, outc_v, b_v, s_sh, sem_a, sem_b, sem_i):
    wid = lax.axis_index("s") * NC + lax.axis_index("c")
    pltpu.sync_copy(b16_hbm, b_v)
    bias = b_v[...]

    def load_idx(ch):
        col = (wid * NCH + ch) * CCH
        ia = pltpu.async_copy(
            xt_hbm.at[pl.ds(0, LA), pl.ds(col, CCH)], idx_a, sem_i)
        ib = pltpu.async_copy(
            xt_hbm.at[pl.ds(LA, LB), pl.ds(col, CCH)], idx_b, sem_i)
        return ia, ib

    def start_gathers():
        ca = pltpu.async_copy(s_sh.at[idx_a.reshape(1, HALFA).at[0]], vals_a,
                              sem_a)
        cb = pltpu.async_copy(s_sh.at[idx_b.reshape(1, HALFB).at[0]], vals_b,
                              sem_b)
        return ca, cb

    ia, ib = load_idx(0)
    sid = lax.axis_index("s")
    spart = SPAD // NS
    pltpu.sync_copy(s_hbm.at[pl.ds(sid * spart, spart)],
                    s_sh.at[pl.ds(sid * spart, spart)])
    plsc.subcore_barrier()
    ia.wait()
    ca = pltpu.async_copy(s_sh.at[idx_a.reshape(1, HALFA).at[0]], vals_a,
                          sem_a)
    ib.wait()
    cb = pltpu.async_copy(s_sh.at[idx_b.reshape(1, HALFB).at[0]], vals_b,
                          sem_b)
    for ch in range(NCH):
        ca.wait()
        cb.wait()
        ci = None
        if ch + 1 < NCH:
            ci = load_idx(ch + 1)
        for g in range(CCH // 16):
            off = g * 16

            def body_a(ll, acc, off=off):
                for u in range(4):
                    acc = acc + vals_a[pl.ds((ll * 4 + u) * CCH + off, 16)]
                return acc

            def body_b(ll, acc, off=off):
                for u in range(4):
                    acc = acc + vals_b[pl.ds((ll * 4 + u) * CCH + off, 16)]
                return acc

            acc = lax.fori_loop(0, LA // 4, body_a,
                                jnp.zeros((16,), jnp.float32))
            acc = lax.fori_loop(0, LB // 4, body_b, acc)
            first = vals_a[pl.ds(off, 16)]
            last = vals_b[pl.ds((LB - 1) * CCH + off, 16)]
            outc_v[pl.ds(off, 16)] = (3.0 * acc - first - last) * INV + bias
        blk = wid * NCH + ch
        pltpu.sync_copy(outc_v, out_hbm.at[pl.ds(blk * CCH, CCH)])
        if ci is not None:
            ci[0].wait()
            ci[1].wait()
            ca, cb = start_gathers()


def kernel(x_in, table, W, b):
    table_t = table.T
    w128 = jnp.broadcast_to(W, (D_N, 128))
    s = _stage1(table_t, w128)
    xt = x_in.astype(jnp.int32).T
    b16 = jnp.broadcast_to(b, (16,))
    pooled = _pool_kernel(xt, s, b16)
    return pooled.reshape(B_N, 1)
